# baseline (speedup 1.0000x reference)
import jax
import jax.numpy as jnp
from jax.experimental import pallas as pl
from jax.experimental.pallas import tpu as pltpu

_B, _C, _H, _W = 32, 384, 64, 64
_HW = _H * _W


def _body(stage_ref, x_ref, e_ref, o_ref):
    del stage_ref
    o_ref[...] = x_ref[...] * e_ref[...]


def kernel(x, stage, embeddings):
    s = jnp.asarray(stage, dtype=jnp.int32).reshape((1,))
    x3 = x.reshape(_B, _C, _HW)
    e3 = embeddings.reshape(embeddings.shape[0], _C, 1)

    grid_spec = pltpu.PrefetchScalarGridSpec(
        num_scalar_prefetch=1,
        grid=(_B,),
        in_specs=[
            pl.BlockSpec((1, _C, _HW), lambda i, st: (i, 0, 0)),
            pl.BlockSpec((1, _C, 1), lambda i, st: (st[0], 0, 0)),
        ],
        out_specs=pl.BlockSpec((1, _C, _HW), lambda i, st: (i, 0, 0)),
    )

    out = pl.pallas_call(
        _body,
        grid_spec=grid_spec,
        out_shape=jax.ShapeDtypeStruct((_B, _C, _HW), jnp.float32),
        compiler_params=pltpu.CompilerParams(
            dimension_semantics=("arbitrary",),
        ),
    )(s, x3, e3)
    return out.reshape(_B, _C, _H, _W)
